# baseline (speedup 1.0000x reference)
import functools

import jax
import jax.numpy as jnp
from jax import lax
from jax.experimental import pallas as pl
from jax.experimental.pallas import tpu as pltpu
from jax.experimental.pallas import tpu_sc as plsc

_NOTE_V = 13
_OCT_V = 12
_EVT_V = 4
_OUT_D = 22
_CVOCAB = _NOTE_V * _OCT_V * _EVT_V
_GPLANES = tuple(range(8)) + (9,)

_NC = 2
_NS = 16
_NW = _NC * _NS
_LANES = 16
_SUB = 32
_MIN = 128
_TS = 8
_HS = 4
_EXTRA = 16


def _sc_body(pos4, pit3, oct3, evt3, vel3, tblT, out4,
             tbl_v, c_v, o_v, e_v, gb0, gb1, sem, osem, bsem,
             L, PD):
    wid = lax.axis_index("s") * _NC + lax.axis_index("c")
    n_stripes = L // _TS
    n_copies = PD * n_stripes
    n_idle_copies = (_NW - n_stripes) * _EXTRA
    n_shared = (n_copies - n_idle_copies) // _NW
    bufs = (gb0, gb1)
    bsems = (bsem, osem)

    pltpu.sync_copy(tblT, tbl_v)

    def bounce(u, buf, bsm, first):
        @pl.when(jnp.logical_not(first))
        def _():
            pltpu.make_async_copy(pos4.at[pl.ds(0, 1)], buf, bsm).wait()
        inc = pltpu.async_copy(pos4.at[pl.ds(u, 1)], buf, bsm)
        inc.wait()
        pltpu.async_copy(buf, out4.at[pl.ds(10 * n_stripes + u, 1)], bsm)

    def drain(buf, bsm):
        pltpu.make_async_copy(pos4.at[pl.ds(0, 1)], buf, bsm).wait()

    @pl.when(wid < n_stripes)
    def _gather_stripe():
        i = wid
        for shalf in range(2):
            l0 = _TS * i + _HS * shalf
            cpc = pltpu.async_copy(pit3.at[pl.ds(l0, _HS)], c_v, sem)
            cpo = pltpu.async_copy(oct3.at[pl.ds(l0, _HS)], o_v, sem)
            cpe = pltpu.async_copy(evt3.at[pl.ds(l0, _HS)], e_v, sem)
            cpc.wait()
            cpo.wait()
            cpe.wait()

            nq = _SUB * (_MIN // _LANES)

            @plsc.parallel_loop(0, nq, unroll=4)
            def _cpass(q):
                j = q >> 3
                ksl = pl.ds((q & 7) * _LANES, _LANES)
                for s in range(_HS):
                    sl = (s, j, ksl)
                    c_v[sl] = (c_v[sl] + _NOTE_V * o_v[sl]
                               + (_NOTE_V * _OCT_V) * e_v[sl])

            cpv = pltpu.async_copy(vel3.at[pl.ds(l0, _HS)], e_v, sem)

            outcp = [None, None]
            for t, f in enumerate(_GPLANES + (8,)):
                gbuf = bufs[t % 2]
                if outcp[t % 2] is not None:
                    outcp[t % 2].wait()

                if f == 8:
                    cpv.wait()

                    @plsc.parallel_loop(0, nq, unroll=4)
                    def _vcopy(q):
                        j = q >> 3
                        ksl = pl.ds((q & 7) * _LANES, _LANES)
                        for s in range(_HS):
                            gbuf[0, j, s, ksl] = plsc.bitcast(
                                e_v[s, j, ksl], jnp.float32)
                else:
                    base = t * _CVOCAB

                    @plsc.parallel_loop(0, nq, unroll=4)
                    def _gpass(q):
                        j = q >> 3
                        ksl = pl.ds((q & 7) * _LANES, _LANES)
                        for s in range(_HS):
                            gbuf[0, j, s, ksl] = plsc.load_gather(
                                tbl_v, [c_v[s, j, ksl] + base])

                outcp[t % 2] = pltpu.async_copy(
                    gbuf.at[:, :, pl.ds(0, _HS), :],
                    out4.at[pl.ds(f * n_stripes + i, 1), :,
                            pl.ds(_HS * shalf, _HS), :], osem)
            outcp[0].wait()
            outcp[1].wait()

    @pl.when(wid >= n_stripes)
    def _idle_copies():
        w0 = (wid - n_stripes) * _EXTRA

        def pair(k, _):
            for b in range(2):
                bounce(w0 + 2 * k + b, bufs[b], bsems[b], k == 0)
            return 0

        lax.fori_loop(0, _EXTRA // 2, pair, 0)
        drain(gb0, bsem)
        drain(gb1, osem)

    @pl.when(wid < n_stripes)
    def _shared_copies():
        def pair(k, _):
            for b in range(2):
                u = n_idle_copies + (2 * k + b) * n_stripes + wid

                @pl.when(u < n_copies)
                def _():
                    bounce(u, bufs[b], bsems[b], k == 0)
            return 0

        lax.fori_loop(0, (n_copies - n_idle_copies + n_stripes - 1)
                      // n_stripes // 2 + 1, pair, 0)
        drain(gb0, bsem)
        drain(gb1, osem)


@jax.jit
def kernel(position, pitch, octave, velocity, note_event_type,
           pitch_table, octave_table, event_type_table):
    B, L, PD = position.shape
    ns = L // _TS
    pos4 = (jnp.transpose(position, (2, 1, 0))
            .reshape(PD, ns, _TS, _SUB, _MIN)
            .transpose(0, 1, 3, 2, 4)
            .reshape(PD * ns, _SUB, _TS, _MIN))
    pit3 = jnp.transpose(pitch, (1, 2, 0)).reshape(L, _SUB, _MIN)
    oct3 = jnp.transpose(octave, (1, 2, 0)).reshape(L, _SUB, _MIN)
    evt3 = jnp.transpose(note_event_type, (1, 2, 0)).reshape(L, _SUB, _MIN)
    vel3 = jax.lax.bitcast_convert_type(
        jnp.transpose(velocity, (1, 2, 0)).reshape(L, _SUB, _MIN), jnp.int32)
    pit3 = pit3.astype(jnp.int32)
    oct3 = oct3.astype(jnp.int32)
    evt3 = evt3.astype(jnp.int32)

    c = jnp.arange(_CVOCAB, dtype=jnp.int32)
    tp = jnp.take(pitch_table, c % _NOTE_V, axis=0)
    to = jnp.take(octave_table, (c // _NOTE_V) % _OCT_V, axis=0)
    te = jnp.take(event_type_table, c // (_NOTE_V * _OCT_V), axis=0)
    tblT = jnp.concatenate([tp, to, te], axis=1).T.reshape(9 * _CVOCAB)

    mesh = plsc.VectorSubcoreMesh(core_axis_name="c", subcore_axis_name="s")
    body = functools.partial(_sc_body, L=L, PD=PD)
    out4 = pl.kernel(
        body,
        out_type=jax.ShapeDtypeStruct((_OUT_D * ns, _SUB, _TS, _MIN),
                                      jnp.float32),
        mesh=mesh,
        compiler_params=pltpu.CompilerParams(needs_layout_passes=False),
        scratch_types=[
            pltpu.VMEM((9 * _CVOCAB,), jnp.float32),
            pltpu.VMEM((_HS, _SUB, _MIN), jnp.int32),
            pltpu.VMEM((_HS, _SUB, _MIN), jnp.int32),
            pltpu.VMEM((_HS, _SUB, _MIN), jnp.int32),
            pltpu.VMEM((1, _SUB, _TS, _MIN), jnp.float32),
            pltpu.VMEM((1, _SUB, _TS, _MIN), jnp.float32),
            pltpu.SemaphoreType.DMA,
            pltpu.SemaphoreType.DMA,
            pltpu.SemaphoreType.DMA,
        ],
    )(pos4, pit3, oct3, evt3, vel3, tblT)
    out = (out4.reshape(_OUT_D, ns, _SUB, _TS, _MIN)
           .transpose(0, 1, 3, 2, 4)
           .reshape(_OUT_D, L, B))
    return jnp.transpose(out, (2, 1, 0))


# speedup vs baseline: 1.0174x; 1.0174x over previous
import functools

import jax
import jax.numpy as jnp
from jax import lax
from jax.experimental import pallas as pl
from jax.experimental.pallas import tpu as pltpu
from jax.experimental.pallas import tpu_sc as plsc

_NOTE_V = 13
_OCT_V = 12
_EVT_V = 4
_OUT_D = 22
_CVOCAB = _NOTE_V * _OCT_V * _EVT_V
_GPLANES = tuple(range(8)) + (9,)

_NC = 2
_NS = 16
_NW = _NC * _NS
_LANES = 16
_SUB = 32
_MIN = 128
_TS = 8
_HS = 4
_EXTRA = 20


def _sc_body(pos4, pit3, oct3, evt3, vel3, tblT, out4,
             tbl_v, c_v, o_v, e_v, gb0, gb1, sem, osem, bsem,
             L, PD):
    wid = lax.axis_index("s") * _NC + lax.axis_index("c")
    n_stripes = L // _TS
    n_copies = PD * n_stripes
    n_idle_copies = (_NW - n_stripes) * _EXTRA
    n_shared = (n_copies - n_idle_copies) // _NW
    bufs = (gb0, gb1)
    bsems = (bsem, osem)

    pltpu.sync_copy(tblT, tbl_v)

    def bounce(u, buf, bsm, first):
        @pl.when(jnp.logical_not(first))
        def _():
            pltpu.make_async_copy(pos4.at[pl.ds(0, 1)], buf, bsm).wait()
        inc = pltpu.async_copy(pos4.at[pl.ds(u, 1)], buf, bsm)
        inc.wait()
        pltpu.async_copy(buf, out4.at[pl.ds(10 * n_stripes + u, 1)], bsm)

    def drain(buf, bsm):
        pltpu.make_async_copy(pos4.at[pl.ds(0, 1)], buf, bsm).wait()

    @pl.when(wid < n_stripes)
    def _gather_stripe():
        i = wid
        for shalf in range(2):
            l0 = _TS * i + _HS * shalf
            cpc = pltpu.async_copy(pit3.at[pl.ds(l0, _HS)], c_v, sem)
            cpo = pltpu.async_copy(oct3.at[pl.ds(l0, _HS)], o_v, sem)
            cpe = pltpu.async_copy(evt3.at[pl.ds(l0, _HS)], e_v, sem)
            cpc.wait()
            cpo.wait()
            cpe.wait()

            nq = _SUB * (_MIN // _LANES)

            @plsc.parallel_loop(0, nq, unroll=4)
            def _cpass(q):
                j = q >> 3
                ksl = pl.ds((q & 7) * _LANES, _LANES)
                for s in range(_HS):
                    sl = (s, j, ksl)
                    c_v[sl] = (c_v[sl] + _NOTE_V * o_v[sl]
                               + (_NOTE_V * _OCT_V) * e_v[sl])

            cpv = pltpu.async_copy(vel3.at[pl.ds(l0, _HS)], e_v, sem)

            outcp = [None, None]
            for t, f in enumerate(_GPLANES + (8,)):
                gbuf = bufs[t % 2]
                if outcp[t % 2] is not None:
                    outcp[t % 2].wait()

                if f == 8:
                    cpv.wait()

                    @plsc.parallel_loop(0, nq, unroll=4)
                    def _vcopy(q):
                        j = q >> 3
                        ksl = pl.ds((q & 7) * _LANES, _LANES)
                        for s in range(_HS):
                            gbuf[0, j, s, ksl] = plsc.bitcast(
                                e_v[s, j, ksl], jnp.float32)
                else:
                    base = t * _CVOCAB

                    @plsc.parallel_loop(0, nq, unroll=4)
                    def _gpass(q):
                        j = q >> 3
                        ksl = pl.ds((q & 7) * _LANES, _LANES)
                        for s in range(_HS):
                            gbuf[0, j, s, ksl] = plsc.load_gather(
                                tbl_v, [c_v[s, j, ksl] + base])

                outcp[t % 2] = pltpu.async_copy(
                    gbuf.at[:, :, pl.ds(0, _HS), :],
                    out4.at[pl.ds(f * n_stripes + i, 1), :,
                            pl.ds(_HS * shalf, _HS), :], osem)
            outcp[0].wait()
            outcp[1].wait()

    @pl.when(wid >= n_stripes)
    def _idle_copies():
        w0 = (wid - n_stripes) * _EXTRA

        def pair(k, _):
            for b in range(2):
                bounce(w0 + 2 * k + b, bufs[b], bsems[b], k == 0)
            return 0

        lax.fori_loop(0, _EXTRA // 2, pair, 0)
        drain(gb0, bsem)
        drain(gb1, osem)

    def _shared_copies():
        def pair(k, _):
            for b in range(2):
                u = n_idle_copies + wid + (2 * k + b) * _NW
                bounce(u, bufs[b], bsems[b], k == 0)
            return 0

        lax.fori_loop(0, n_shared // 2, pair, 0)
        drain(gb0, bsem)
        drain(gb1, osem)
        for r in range(n_shared % 2):
            u = n_idle_copies + wid + (n_shared - 1) * _NW
            inc = pltpu.async_copy(pos4.at[pl.ds(u, 1)], gb0, bsem)
            inc.wait()
            pltpu.sync_copy(gb0, out4.at[pl.ds(10 * n_stripes + u, 1)])

    _shared_copies()


@jax.jit
def kernel(position, pitch, octave, velocity, note_event_type,
           pitch_table, octave_table, event_type_table):
    B, L, PD = position.shape
    ns = L // _TS
    pos4 = (jnp.transpose(position, (2, 1, 0))
            .reshape(PD, ns, _TS, _SUB, _MIN)
            .transpose(0, 1, 3, 2, 4)
            .reshape(PD * ns, _SUB, _TS, _MIN))
    pit3 = jnp.transpose(pitch, (1, 2, 0)).reshape(L, _SUB, _MIN)
    oct3 = jnp.transpose(octave, (1, 2, 0)).reshape(L, _SUB, _MIN)
    evt3 = jnp.transpose(note_event_type, (1, 2, 0)).reshape(L, _SUB, _MIN)
    vel3 = jax.lax.bitcast_convert_type(
        jnp.transpose(velocity, (1, 2, 0)).reshape(L, _SUB, _MIN), jnp.int32)
    pit3 = pit3.astype(jnp.int32)
    oct3 = oct3.astype(jnp.int32)
    evt3 = evt3.astype(jnp.int32)

    c = jnp.arange(_CVOCAB, dtype=jnp.int32)
    tp = jnp.take(pitch_table, c % _NOTE_V, axis=0)
    to = jnp.take(octave_table, (c // _NOTE_V) % _OCT_V, axis=0)
    te = jnp.take(event_type_table, c // (_NOTE_V * _OCT_V), axis=0)
    tblT = jnp.concatenate([tp, to, te], axis=1).T.reshape(9 * _CVOCAB)

    mesh = plsc.VectorSubcoreMesh(core_axis_name="c", subcore_axis_name="s")
    body = functools.partial(_sc_body, L=L, PD=PD)
    out4 = pl.kernel(
        body,
        out_type=jax.ShapeDtypeStruct((_OUT_D * ns, _SUB, _TS, _MIN),
                                      jnp.float32),
        mesh=mesh,
        compiler_params=pltpu.CompilerParams(needs_layout_passes=False),
        scratch_types=[
            pltpu.VMEM((9 * _CVOCAB,), jnp.float32),
            pltpu.VMEM((_HS, _SUB, _MIN), jnp.int32),
            pltpu.VMEM((_HS, _SUB, _MIN), jnp.int32),
            pltpu.VMEM((_HS, _SUB, _MIN), jnp.int32),
            pltpu.VMEM((1, _SUB, _TS, _MIN), jnp.float32),
            pltpu.VMEM((1, _SUB, _TS, _MIN), jnp.float32),
            pltpu.SemaphoreType.DMA,
            pltpu.SemaphoreType.DMA,
            pltpu.SemaphoreType.DMA,
        ],
    )(pos4, pit3, oct3, evt3, vel3, tblT)
    out = (out4.reshape(_OUT_D, ns, _SUB, _TS, _MIN)
           .transpose(0, 1, 3, 2, 4)
           .reshape(_OUT_D, L, B))
    return jnp.transpose(out, (2, 1, 0))
